# Initial kernel scaffold; baseline (speedup 1.0000x reference)
import jax
import jax.numpy as jnp
from jax.experimental import pallas as pl


def kernel(user_ids, item_ids, Ug, Ig, Um, Im, W1, b1, W2, b2, W3, b3, Wf, bf):
    raise NotImplementedError("write your pallas kernel here")


# speedup vs baseline: 2.3670x; 2.3670x over previous
import functools

import jax
import jax.numpy as jnp
from jax import lax
from jax.experimental import pallas as pl
from jax.experimental.pallas import tpu as pltpu
from jax.experimental.pallas import tpu_sc as plsc

B = 16384
EMB = 128
NW = 32
BPW = B // NW
CHUNK = 128
NCHUNK = BPW // CHUNK


def _sc_gather4(uids2, iids2, Ug, Ig, Um, Im):
    mesh = plsc.VectorSubcoreMesh(core_axis_name="c", subcore_axis_name="s")

    def k_body(u_hbm, i_hbm, ug_hbm, ig_hbm, um_hbm, im_hbm,
               out_ug, out_ig, out_um, out_im,
               uidx, iidx, buf0, buf1, sem0, sem1):
        wid = lax.axis_index("s") * 2 + lax.axis_index("c")
        idx_row0 = wid * NCHUNK
        pltpu.sync_copy(u_hbm.at[pl.ds(idx_row0, NCHUNK)], uidx)
        pltpu.sync_copy(i_hbm.at[pl.ds(idx_row0, NCHUNK)], iidx)

        jobs = []
        for tbl, idx, out in ((ug_hbm, uidx, out_ug), (ig_hbm, iidx, out_ig),
                              (um_hbm, uidx, out_um), (im_hbm, iidx, out_im)):
            for c in range(NCHUNK):
                jobs.append((tbl, idx, c, out))
        bufs = (buf0, buf1)
        sems = (sem0, sem1)
        row0 = wid * BPW
        pending = None
        for j, (tbl, idx, c, out) in enumerate(jobs):
            cp = pltpu.async_copy(tbl.at[idx.at[c]], bufs[j % 2], sems[j % 2])
            if pending is not None:
                pj, pcp, pout, pc = pending
                pcp.wait()
                pltpu.sync_copy(bufs[pj % 2],
                                pout.at[pl.ds(row0 + pc * CHUNK, CHUNK)])
            pending = (j, cp, out, c)
        pj, pcp, pout, pc = pending
        pcp.wait()
        pltpu.sync_copy(bufs[pj % 2], pout.at[pl.ds(row0 + pc * CHUNK, CHUNK)])

    return pl.kernel(
        k_body,
        out_type=[jax.ShapeDtypeStruct((B, EMB), jnp.float32)] * 4,
        mesh=mesh,
        scratch_types=[
            pltpu.VMEM((NCHUNK, CHUNK), jnp.int32),
            pltpu.VMEM((NCHUNK, CHUNK), jnp.int32),
            pltpu.VMEM((CHUNK, EMB), jnp.float32),
            pltpu.VMEM((CHUNK, EMB), jnp.float32),
            pltpu.SemaphoreType.DMA,
            pltpu.SemaphoreType.DMA,
        ],
    )(uids2, iids2, Ug, Ig, Um, Im)


def _mlp_body(ug_r, ig_r, um_r, im_r, w1a_r, w1b_r, b1_r, w2_r, b2_r,
              w3_r, b3_r, wfa_r, wfb_r, bf_r, out_r):
    f32 = jnp.float32
    h1 = um_r[...] @ w1a_r[...] + im_r[...] @ w1b_r[...] + b1_r[...]
    h1 = jnp.maximum(h1, 0.0)
    h2 = jnp.maximum(h1 @ w2_r[...] + b2_r[...], 0.0)
    h3 = jnp.maximum(h2 @ w3_r[...] + b3_r[...], 0.0)
    gmf = ug_r[...] * ig_r[...]
    z = gmf @ wfa_r[...] + h3 @ wfb_r[...] + bf_r[...]
    out_r[...] = 1.0 / (1.0 + jnp.exp(-z))


def _tc_mlp(ug, ig, um, im, W1a, W1b, b1, W2, b2, W3, b3, Wfa, Wfb, bf):
    R = 1024
    grid = (B // R,)
    row_spec = pl.BlockSpec((R, EMB), lambda i: (i, 0))

    def fixed(shape):
        return pl.BlockSpec(shape, lambda i: tuple(0 for _ in shape))

    return pl.pallas_call(
        _mlp_body,
        grid=grid,
        in_specs=[
            row_spec, row_spec, row_spec, row_spec,
            fixed((EMB, 256)), fixed((EMB, 256)), fixed((1, 256)),
            fixed((256, 128)), fixed((1, 128)),
            fixed((128, 64)), fixed((1, 64)),
            fixed((EMB, 1)), fixed((64, 1)), fixed((1, 1)),
        ],
        out_specs=pl.BlockSpec((R, 1), lambda i: (i, 0)),
        out_shape=jax.ShapeDtypeStruct((B, 1), jnp.float32),
    )(ug, ig, um, im, W1a, W1b, b1, W2, b2, W3, b3, Wfa, Wfb, bf)


def kernel(user_ids, item_ids, Ug, Ig, Um, Im, W1, b1, W2, b2, W3, b3, Wf, bf):
    uids2 = user_ids.astype(jnp.int32).reshape(B // CHUNK, CHUNK)
    iids2 = item_ids.astype(jnp.int32).reshape(B // CHUNK, CHUNK)
    ug, ig, um, im = _sc_gather4(uids2, iids2, Ug, Ig, Um, Im)
    W1a, W1b = W1[:EMB], W1[EMB:]
    Wfa, Wfb = Wf[:EMB], Wf[EMB:]
    return _tc_mlp(ug, ig, um, im,
                   W1a, W1b, b1.reshape(1, -1),
                   W2, b2.reshape(1, -1), W3, b3.reshape(1, -1),
                   Wfa, Wfb, bf.reshape(1, 1))
